# baseline (speedup 1.0000x reference)
import functools

import jax
import jax.numpy as jnp
from jax import lax
from jax.experimental import pallas as pl
from jax.experimental.pallas import tpu as pltpu
from jax.experimental.pallas import tpu_sc as plsc

N = 10000
E = 320000
D = 128
NEG_SLOPE = 0.2

NC = 2
NS = 16
L = 16
NW = NC * NS
C = 80
EWP = 10080
EP = NW * EWP
NCHUNK = EWP // C
NPAIR = NCHUNK // 2
RPT8 = 624
DEN_R = 79


def _ab_body(h_ref, w_ref, o_ref):
    o_ref[...] = jnp.dot(h_ref[...], w_ref[...],
                         preferred_element_type=jnp.float32)


def _ab_call(h, w_pad):
    blk = 1000
    return pl.pallas_call(
        _ab_body,
        grid=(N // blk,),
        in_specs=[
            pl.BlockSpec((blk, D), lambda i: (i, 0)),
            pl.BlockSpec((D, 128), lambda i: (0, 0)),
        ],
        out_specs=pl.BlockSpec((blk, 128), lambda i: (i, 0)),
        out_shape=jax.ShapeDtypeStruct((N, 128), jnp.float32),
    )(h, w_pad)


def _sc_body(h_hbm, ei_hbm, a_hbm, b_hbm, part_hbm, den_hbm,
             src0, dst0, src1, dst1, a_v, b_v, den_v, rows0, rows1, acc,
             gsem0, gsem1, isem0, isem1):
    cid = lax.axis_index("c")
    sid = lax.axis_index("s")
    wid = sid * NC + cid
    ebase = wid * EWP

    pltpu.sync_copy(a_hbm, a_v)
    pltpu.sync_copy(b_hbm, b_v)

    def _zden(i, carry):
        for j in range(128 // L):
            den_v[i, pl.ds(j * L, L)] = jnp.zeros((L,), jnp.float32)
        return carry
    lax.fori_loop(0, DEN_R, _zden, 0)

    def _zrow(i, carry):
        for j in range(D // L):
            rows0[i, pl.ds(j * L, L)] = jnp.zeros((L,), jnp.float32)
        return carry
    lax.fori_loop(0, C, _zrow, 0)

    base = sid * RPT8
    zcps = [pltpu.async_copy(rows0, acc.at[pl.ds(base + kk * C, C)], gsem0)
            for kk in range(RPT8 // C)]
    zcps.append(pltpu.async_copy(rows0.at[pl.ds(0, RPT8 % C)],
                                 acc.at[pl.ds(base + (RPT8 // C) * C,
                                              RPT8 % C)], gsem0))

    @pl.when(sid == NS - 1)
    def _tail_zero():
        pltpu.sync_copy(rows0.at[pl.ds(0, N - NS * RPT8)],
                        acc.at[pl.ds(NS * RPT8, N - NS * RPT8)])
    for cp in zcps:
        cp.wait()
    plsc.subcore_barrier()

    def _process(ci, src_r, dst_r, rows_r):
        for g in range(C // L):
            sv = src_r[pl.ds(g * L, L)]
            dv = dst_r[pl.ds(g * L, L)]
            e = plsc.load_gather(a_v, [sv]) + plsc.load_gather(b_v, [dv])
            e = jnp.where(e >= 0, e, NEG_SLOPE * e)
            p = jnp.exp(e)
            gid = ebase + ci * C + g * L + lax.iota(jnp.int32, L)
            p = jnp.where(gid < E, p, 0.0)
            plsc.addupdate_scatter(
                den_v, [lax.shift_right_logical(dv, 7),
                        jnp.bitwise_and(dv, 127)], p)
            for l in range(L):
                pi = p[l]
                i = g * L + l
                for j in range(D // L):
                    rows_r[i, pl.ds(j * L, L)] = (
                        rows_r[i, pl.ds(j * L, L)] * pi)
        pltpu.sync_copy(rows_r, acc.at[dst_r], add=True)

    pltpu.async_copy(ei_hbm.at[0, pl.ds(ebase, C)], src0, isem0)
    pltpu.async_copy(ei_hbm.at[1, pl.ds(ebase, C)], dst0, isem0)
    pltpu.make_async_copy(ei_hbm.at[0, pl.ds(0, C)], src0, isem0).wait()
    pltpu.make_async_copy(ei_hbm.at[1, pl.ds(0, C)], dst0, isem0).wait()
    pltpu.async_copy(ei_hbm.at[0, pl.ds(ebase + C, C)], src1, isem1)
    pltpu.async_copy(ei_hbm.at[1, pl.ds(ebase + C, C)], dst1, isem1)
    pltpu.async_copy(h_hbm.at[src0], rows0, gsem0)

    def _pair(k, carry):
        c0 = 2 * k
        pltpu.make_async_copy(ei_hbm.at[0, pl.ds(0, C)], src1, isem1).wait()
        pltpu.make_async_copy(ei_hbm.at[1, pl.ds(0, C)], dst1, isem1).wait()
        pltpu.async_copy(h_hbm.at[src1], rows1, gsem1)
        pltpu.make_async_copy(h_hbm.at[pl.ds(0, C)], rows0, gsem0).wait()
        _process(c0, src0, dst0, rows0)

        @pl.when(k < NPAIR - 1)
        def _pf0():
            pltpu.async_copy(ei_hbm.at[0, pl.ds(ebase + (c0 + 2) * C, C)],
                             src0, isem0)
            pltpu.async_copy(ei_hbm.at[1, pl.ds(ebase + (c0 + 2) * C, C)],
                             dst0, isem0)

        pltpu.make_async_copy(h_hbm.at[pl.ds(0, C)], rows1, gsem1).wait()
        _process(c0 + 1, src1, dst1, rows1)

        @pl.when(k < NPAIR - 1)
        def _pf1():
            pltpu.async_copy(ei_hbm.at[0, pl.ds(ebase + (c0 + 3) * C, C)],
                             src1, isem1)
            pltpu.async_copy(ei_hbm.at[1, pl.ds(ebase + (c0 + 3) * C, C)],
                             dst1, isem1)
            pltpu.make_async_copy(ei_hbm.at[0, pl.ds(0, C)], src0,
                                  isem0).wait()
            pltpu.make_async_copy(ei_hbm.at[1, pl.ds(0, C)], dst0,
                                  isem0).wait()
            pltpu.async_copy(h_hbm.at[src0], rows0, gsem0)
        return carry
    lax.fori_loop(0, NPAIR, _pair, 0)

    plsc.subcore_barrier()

    pltpu.sync_copy(acc.at[pl.ds(base, RPT8)],
                    part_hbm.at[cid, pl.ds(base, RPT8)])

    @pl.when(sid == NS - 1)
    def _tail_out():
        pltpu.sync_copy(acc.at[pl.ds(NS * RPT8, N - NS * RPT8)],
                        part_hbm.at[cid, pl.ds(NS * RPT8, N - NS * RPT8)])

    pltpu.sync_copy(den_v, den_hbm.at[wid])


def _sc_call(h, ei_p, a, b):
    mesh = plsc.VectorSubcoreMesh(core_axis_name="c", subcore_axis_name="s",
                                  num_cores=NC, num_subcores=NS)
    fn = pl.kernel(
        _sc_body,
        out_type=(
            jax.ShapeDtypeStruct((NC, N, D), jnp.float32),
            jax.ShapeDtypeStruct((NW, DEN_R, 128), jnp.float32),
        ),
        mesh=mesh,
        compiler_params=pltpu.CompilerParams(needs_layout_passes=False,
                                             use_tc_tiling_on_sc=False),
        scratch_types=(
            pltpu.VMEM((C,), jnp.int32),
            pltpu.VMEM((C,), jnp.int32),
            pltpu.VMEM((C,), jnp.int32),
            pltpu.VMEM((C,), jnp.int32),
            pltpu.VMEM((N,), jnp.float32),
            pltpu.VMEM((N,), jnp.float32),
            pltpu.VMEM((DEN_R, 128), jnp.float32),
            pltpu.VMEM((C, D), jnp.float32),
            pltpu.VMEM((C, D), jnp.float32),
            pltpu.VMEM_SHARED((N, D), jnp.float32),
            pltpu.SemaphoreType.DMA,
            pltpu.SemaphoreType.DMA,
            pltpu.SemaphoreType.DMA,
            pltpu.SemaphoreType.DMA,
        ),
    )
    return fn(h, ei_p, a, b)


def _fin_body(p0_ref, p1_ref, d_ref, o_ref):
    s = p0_ref[0] + p1_ref[0]
    den = jnp.sum(d_ref[...], axis=1)
    o_ref[...] = s / jnp.maximum(den, 1e-9)[:, None]


def _fin_call(partials, denoms_t):
    blk = 400
    return pl.pallas_call(
        _fin_body,
        grid=(N // blk,),
        in_specs=[
            pl.BlockSpec((1, blk, D), lambda i: (0, i, 0)),
            pl.BlockSpec((1, blk, D), lambda i: (1, i, 0)),
            pl.BlockSpec((blk, NW), lambda i: (i, 0)),
        ],
        out_specs=pl.BlockSpec((blk, D), lambda i: (i, 0)),
        out_shape=jax.ShapeDtypeStruct((N, D), jnp.float32),
    )(partials, partials, denoms_t)


@jax.jit
def kernel(h, edge_index, W_att):
    w_row = W_att[0]
    w_pad = jnp.zeros((D, 128), jnp.float32)
    w_pad = w_pad.at[:, 0].set(w_row[:D]).at[:, 1].set(w_row[D:])
    ab = _ab_call(h, w_pad)
    a = ab[:, 0]
    b = ab[:, 1]
    ei_p = jnp.pad(edge_index, ((0, 0), (0, EP - E)))
    partials, denoms = _sc_call(h, ei_p, a, b)
    den_t = denoms.reshape(NW, DEN_R * 128)[:, :N].T
    return _fin_call(partials, den_t)
